# baseline (speedup 1.0000x reference)
import functools

import jax
import jax.numpy as jnp
from jax import lax
from jax.experimental import pallas as pl
from jax.experimental.pallas import tpu as pltpu
from jax.experimental.pallas import tpu_sc as plsc

M = 16384
E = 128


CHUNK = 128


@functools.lru_cache(maxsize=None)
def _build_embed_kernel():
    info = plsc.get_sparse_core_info()
    nw = info.num_cores * info.num_subcores
    b_per_w = M // nw
    nchunks = b_per_w // CHUNK

    mesh = plsc.VectorSubcoreMesh(core_axis_name="c", subcore_axis_name="s")

    @functools.partial(
        pl.kernel,
        mesh=mesh,
        out_type=jax.ShapeDtypeStruct((M, E), jnp.float32),
        scratch_types=[
            pltpu.VMEM((b_per_w,), jnp.int32),
            pltpu.VMEM((b_per_w, E), jnp.float32),
        ]
        + [pltpu.SemaphoreType.DMA] * nchunks
        + [pltpu.SemaphoreType.DMA],
    )
    def embed(idx_hbm, table_hbm, out_hbm, idx_v, rows_v, *sems):
        gsems, ssem = sems[:nchunks], sems[nchunks]
        wid = lax.axis_index("s") * info.num_cores + lax.axis_index("c")
        base = wid * b_per_w
        pltpu.sync_copy(idx_hbm.at[pl.ds(base, b_per_w)], idx_v)
        gathers = []
        for c in range(nchunks):
            gathers.append(
                pltpu.async_copy(
                    table_hbm.at[idx_v.at[pl.ds(c * CHUNK, CHUNK)]],
                    rows_v.at[pl.ds(c * CHUNK, CHUNK)],
                    gsems[c],
                )
            )
        stores = []
        for c in range(nchunks):
            gathers[c].wait()
            stores.append(
                pltpu.async_copy(
                    rows_v.at[pl.ds(c * CHUNK, CHUNK)],
                    out_hbm.at[pl.ds(base + c * CHUNK, CHUNK)],
                    ssem,
                )
            )
        for s in stores:
            s.wait()

    return embed


def kernel(input, embed_weight):
    idx = input.astype(jnp.int32)
    return _build_embed_kernel()(idx, embed_weight)
